# baseline (speedup 1.0000x reference)
import functools

import jax
import jax.numpy as jnp
from jax import lax
from jax.experimental import pallas as pl
from jax.experimental.pallas import tpu as pltpu
from jax.experimental.pallas import tpu_sc as plsc

NC = 2
NS = 16
NW = NC * NS
CH = 128
K0_FRAC = 0.30


def _spmm_sc(z, src_t, dst_t, zeros_slab, n1, k0, k1):
    d = z.shape[1]
    rt = n1 // NS
    kmax = src_t.shape[1]

    mesh = plsc.VectorSubcoreMesh(core_axis_name="c", subcore_axis_name="s")

    @functools.partial(
        pl.kernel,
        out_type=jax.ShapeDtypeStruct((NC, n1, d), jnp.float32),
        mesh=mesh,
        scratch_types=[
            pltpu.VMEM((kmax, CH), jnp.int32),
            pltpu.VMEM((kmax, CH), jnp.int32),
            pltpu.VMEM((CH, d), jnp.float32),
            pltpu.VMEM_SHARED((n1, d), jnp.float32),
        ],
    )
    def spmm(z_hbm, src_hbm, dst_hbm, zeros_hbm, out_hbm,
             idx_s, idx_d, buf, acc):
        c = lax.axis_index("c")
        s = lax.axis_index("s")
        w = s * NC + c
        my_k = jnp.where(c == 0, k0, k1)

        pltpu.sync_copy(zeros_hbm, acc.at[pl.ds(s * rt, rt)])
        pltpu.sync_copy(src_hbm.at[w], idx_s)
        pltpu.sync_copy(dst_hbm.at[w], idx_d)
        plsc.subcore_barrier()

        def body(j, carry):
            pltpu.sync_copy(z_hbm.at[idx_s.at[j]], buf)
            pltpu.sync_copy(buf, acc.at[idx_d.at[j]], add=True)
            return carry

        lax.fori_loop(0, my_k, body, 0)
        plsc.subcore_barrier()
        pltpu.sync_copy(acc.at[pl.ds(s * rt, rt)],
                        out_hbm.at[c, pl.ds(s * rt, rt)])

    return spmm(z, src_t, dst_t, zeros_slab)


def _dense_layer(parts, w_mat, bias, skip, block_rows=1264):
    n1, d = parts.shape[1], parts.shape[2]
    grid = n1 // block_rows
    has_skip = skip is not None

    def body(p_ref, w_ref, b_ref, *rest):
        if has_skip:
            skip_ref, out_ref = rest
        else:
            (out_ref,) = rest
        agg = p_ref[0] + p_ref[1]
        h = jnp.maximum(
            jnp.dot(agg, w_ref[...], preferred_element_type=jnp.float32)
            + b_ref[...], 0.0)
        if has_skip:
            h = h + skip_ref[...]
        out_ref[...] = h

    in_specs = [
        pl.BlockSpec((NC, block_rows, d), lambda i: (0, i, 0)),
        pl.BlockSpec((d, d), lambda i: (0, 0)),
        pl.BlockSpec((1, d), lambda i: (0, 0)),
    ]
    args = [parts, w_mat, bias.reshape(1, d)]
    if has_skip:
        in_specs.append(pl.BlockSpec((block_rows, d), lambda i: (i, 0)))
        args.append(skip)

    return pl.pallas_call(
        body,
        grid=(grid,),
        in_specs=in_specs,
        out_specs=pl.BlockSpec((block_rows, d), lambda i: (i, 0)),
        out_shape=jax.ShapeDtypeStruct((n1, d), jnp.float32),
    )(*args)


def _layer_norm(h, gamma, beta, block_rows=1000):
    n, d = h.shape
    grid = n // block_rows

    def body(h_ref, g_ref, b_ref, out_ref):
        x = h_ref[...]
        mu = jnp.mean(x, axis=1, keepdims=True)
        var = jnp.mean((x - mu) ** 2, axis=1, keepdims=True)
        out_ref[...] = (x - mu) * lax.rsqrt(var + 1e-5) * g_ref[...] + b_ref[...]

    return pl.pallas_call(
        body,
        grid=(grid,),
        in_specs=[
            pl.BlockSpec((block_rows, d), lambda i: (i, 0)),
            pl.BlockSpec((1, d), lambda i: (0, 0)),
            pl.BlockSpec((1, d), lambda i: (0, 0)),
        ],
        out_specs=pl.BlockSpec((block_rows, d), lambda i: (i, 0)),
        out_shape=jax.ShapeDtypeStruct((n, d), jnp.float32),
    )(h, gamma.reshape(1, d), beta.reshape(1, d))


def kernel(feat, edge_index, W0, b0, Ws, bs, W1, b1, gamma, beta):
    n, d = feat.shape
    e = edge_index.shape[1]

    n1 = -(-(n + 1) // 1264) * 1264

    kt = -(-e // (NS * CH))
    k0 = int(kt * K0_FRAC + 0.5)
    k1 = kt - k0
    kmax = max(k0, k1)

    src = edge_index[0]
    dst = edge_index[1]
    pad = NS * kt * CH - e
    src_p = jnp.concatenate([src, jnp.zeros((pad,), jnp.int32)])
    dst_p = jnp.concatenate([dst, jnp.full((pad,), n, jnp.int32)])

    def split(flat, fill):
        per_tile = flat.reshape(NS, kt, CH)
        c0 = per_tile[:, :k0]
        c1 = per_tile[:, k0:]
        if k0 < kmax:
            c0 = jnp.concatenate(
                [c0, jnp.full((NS, kmax - k0, CH), fill, jnp.int32)], axis=1)
        if k1 < kmax:
            c1 = jnp.concatenate(
                [c1, jnp.full((NS, kmax - k1, CH), fill, jnp.int32)], axis=1)
        return jnp.stack([c0, c1], axis=1).reshape(NW, kmax, CH)

    src_t = split(src_p, 0)
    dst_t = split(dst_p, n)
    zeros_slab = jnp.zeros((n1 // NS, d), jnp.float32)

    parts = _spmm_sc(feat, src_t, dst_t, zeros_slab, n1, k0, k1)
    h = _dense_layer(parts, W0, b0, None)
    for _ in range(3):
        parts = _spmm_sc(h, src_t, dst_t, zeros_slab, n1, k0, k1)
        h = _dense_layer(parts, Ws, bs, h)
    parts = _spmm_sc(h, src_t, dst_t, zeros_slab, n1, k0, k1)
    h = _dense_layer(parts, W1, b1, None)
    return _layer_norm(h[:n], gamma, beta)


# speedup vs baseline: 1.1831x; 1.1831x over previous
import functools

import jax
import jax.numpy as jnp
from jax import lax
from jax.experimental import pallas as pl
from jax.experimental.pallas import tpu as pltpu
from jax.experimental.pallas import tpu_sc as plsc

NC = 2
NS = 16
NW = NC * NS
CH = 128
K0_FRAC = 0.45


def _spmm_sc(z, src_t, dst_t, zeros_slab, n1, k0, k1):
    d = z.shape[1]
    rt = n1 // NS
    kmax = src_t.shape[1]

    mesh = plsc.VectorSubcoreMesh(core_axis_name="c", subcore_axis_name="s")

    @functools.partial(
        pl.kernel,
        out_type=jax.ShapeDtypeStruct((NC, n1, d), jnp.float32),
        mesh=mesh,
        scratch_types=[
            pltpu.VMEM((kmax, CH), jnp.int32),
            pltpu.VMEM((kmax, CH), jnp.int32),
            pltpu.VMEM((CH, d), jnp.float32),
            pltpu.VMEM_SHARED((n1, d), jnp.float32),
        ],
    )
    def spmm(z_hbm, src_hbm, dst_hbm, zeros_hbm, out_hbm,
             idx_s, idx_d, buf, acc):
        c = lax.axis_index("c")
        s = lax.axis_index("s")
        w = s * NC + c
        my_k = jnp.where(c == 0, k0, k1)

        pltpu.sync_copy(zeros_hbm, acc.at[pl.ds(s * rt, rt)])
        pltpu.sync_copy(src_hbm.at[w], idx_s)
        pltpu.sync_copy(dst_hbm.at[w], idx_d)
        plsc.subcore_barrier()

        def body(j, carry):
            pltpu.sync_copy(z_hbm.at[idx_s.at[j]], buf)
            pltpu.sync_copy(buf, acc.at[idx_d.at[j]], add=True)
            return carry

        lax.fori_loop(0, my_k, body, 0)
        plsc.subcore_barrier()
        pltpu.sync_copy(acc.at[pl.ds(s * rt, rt)],
                        out_hbm.at[c, pl.ds(s * rt, rt)])

    return spmm(z, src_t, dst_t, zeros_slab)


def _dense_layer(parts, w_mat, bias, skip, block_rows=1264):
    n1, d = parts.shape[1], parts.shape[2]
    grid = n1 // block_rows
    has_skip = skip is not None

    def body(p_ref, w_ref, b_ref, *rest):
        if has_skip:
            skip_ref, out_ref = rest
        else:
            (out_ref,) = rest
        agg = p_ref[0] + p_ref[1]
        h = jnp.maximum(
            jnp.dot(agg, w_ref[...], preferred_element_type=jnp.float32)
            + b_ref[...], 0.0)
        if has_skip:
            h = h + skip_ref[...]
        out_ref[...] = h

    in_specs = [
        pl.BlockSpec((NC, block_rows, d), lambda i: (0, i, 0)),
        pl.BlockSpec((d, d), lambda i: (0, 0)),
        pl.BlockSpec((1, d), lambda i: (0, 0)),
    ]
    args = [parts, w_mat, bias.reshape(1, d)]
    if has_skip:
        in_specs.append(pl.BlockSpec((block_rows, d), lambda i: (i, 0)))
        args.append(skip)

    return pl.pallas_call(
        body,
        grid=(grid,),
        in_specs=in_specs,
        out_specs=pl.BlockSpec((block_rows, d), lambda i: (i, 0)),
        out_shape=jax.ShapeDtypeStruct((n1, d), jnp.float32),
    )(*args)


def _layer_norm(h, gamma, beta, block_rows=1000):
    n, d = h.shape
    grid = n // block_rows

    def body(h_ref, g_ref, b_ref, out_ref):
        x = h_ref[...]
        mu = jnp.mean(x, axis=1, keepdims=True)
        var = jnp.mean((x - mu) ** 2, axis=1, keepdims=True)
        out_ref[...] = (x - mu) * lax.rsqrt(var + 1e-5) * g_ref[...] + b_ref[...]

    return pl.pallas_call(
        body,
        grid=(grid,),
        in_specs=[
            pl.BlockSpec((block_rows, d), lambda i: (i, 0)),
            pl.BlockSpec((1, d), lambda i: (0, 0)),
            pl.BlockSpec((1, d), lambda i: (0, 0)),
        ],
        out_specs=pl.BlockSpec((block_rows, d), lambda i: (i, 0)),
        out_shape=jax.ShapeDtypeStruct((n, d), jnp.float32),
    )(h, gamma.reshape(1, d), beta.reshape(1, d))


def kernel(feat, edge_index, W0, b0, Ws, bs, W1, b1, gamma, beta):
    n, d = feat.shape
    e = edge_index.shape[1]

    n1 = -(-(n + 1) // 1264) * 1264

    kt = -(-e // (NS * CH))
    k0 = int(kt * K0_FRAC + 0.5)
    k1 = kt - k0
    kmax = max(k0, k1)

    src = edge_index[0]
    dst = edge_index[1]
    pad = NS * kt * CH - e
    src_p = jnp.concatenate([src, jnp.zeros((pad,), jnp.int32)])
    dst_p = jnp.concatenate([dst, jnp.full((pad,), n, jnp.int32)])

    def split(flat, fill):
        per_tile = flat.reshape(NS, kt, CH)
        c0 = per_tile[:, :k0]
        c1 = per_tile[:, k0:]
        if k0 < kmax:
            c0 = jnp.concatenate(
                [c0, jnp.full((NS, kmax - k0, CH), fill, jnp.int32)], axis=1)
        if k1 < kmax:
            c1 = jnp.concatenate(
                [c1, jnp.full((NS, kmax - k1, CH), fill, jnp.int32)], axis=1)
        return jnp.stack([c0, c1], axis=1).reshape(NW, kmax, CH)

    src_t = split(src_p, 0)
    dst_t = split(dst_p, n)
    zeros_slab = jnp.zeros((n1 // NS, d), jnp.float32)

    parts = _spmm_sc(feat, src_t, dst_t, zeros_slab, n1, k0, k1)
    h = _dense_layer(parts, W0, b0, None)
    for _ in range(3):
        parts = _spmm_sc(h, src_t, dst_t, zeros_slab, n1, k0, k1)
        h = _dense_layer(parts, Ws, bs, h)
    parts = _spmm_sc(h, src_t, dst_t, zeros_slab, n1, k0, k1)
    h = _dense_layer(parts, W1, b1, None)
    return _layer_norm(h[:n], gamma, beta)


# speedup vs baseline: 1.2665x; 1.0705x over previous
import functools

import jax
import jax.numpy as jnp
from jax import lax
from jax.experimental import pallas as pl
from jax.experimental.pallas import tpu as pltpu
from jax.experimental.pallas import tpu_sc as plsc

NC = 2
NS = 16
NW = NC * NS
CH = 128
K0_FRAC = 0.50


def _spmm_sc(z, src_t, dst_t, zeros_slab, n1, k0, k1):
    d = z.shape[1]
    rt = n1 // NS
    kmax = src_t.shape[1]

    mesh = plsc.VectorSubcoreMesh(core_axis_name="c", subcore_axis_name="s")

    @functools.partial(
        pl.kernel,
        out_type=jax.ShapeDtypeStruct((NC, n1, d), jnp.float32),
        mesh=mesh,
        scratch_types=[
            pltpu.VMEM((kmax, CH), jnp.int32),
            pltpu.VMEM((kmax, CH), jnp.int32),
            pltpu.VMEM((CH, d), jnp.float32),
            pltpu.VMEM_SHARED((n1, d), jnp.float32),
        ],
    )
    def spmm(z_hbm, src_hbm, dst_hbm, zeros_hbm, out_hbm,
             idx_s, idx_d, buf, acc):
        c = lax.axis_index("c")
        s = lax.axis_index("s")
        w = s * NC + c
        my_k = jnp.where(c == 0, k0, k1)

        pltpu.sync_copy(zeros_hbm, acc.at[pl.ds(s * rt, rt)])
        pltpu.sync_copy(src_hbm.at[w], idx_s)
        pltpu.sync_copy(dst_hbm.at[w], idx_d)
        plsc.subcore_barrier()

        def body(j, carry):
            pltpu.sync_copy(z_hbm.at[idx_s.at[j]], buf)
            pltpu.sync_copy(buf, acc.at[idx_d.at[j]], add=True)
            return carry

        lax.fori_loop(0, my_k, body, 0)
        plsc.subcore_barrier()
        pltpu.sync_copy(acc.at[pl.ds(s * rt, rt)],
                        out_hbm.at[c, pl.ds(s * rt, rt)])

    return spmm(z, src_t, dst_t, zeros_slab)


def _dense_layer(parts, w_mat, bias, skip, block_rows=1264):
    n1, d = parts.shape[1], parts.shape[2]
    grid = n1 // block_rows
    has_skip = skip is not None

    def body(p_ref, w_ref, b_ref, *rest):
        if has_skip:
            skip_ref, out_ref = rest
        else:
            (out_ref,) = rest
        agg = p_ref[0] + p_ref[1]
        h = jnp.maximum(
            jnp.dot(agg, w_ref[...], preferred_element_type=jnp.float32)
            + b_ref[...], 0.0)
        if has_skip:
            h = h + skip_ref[...]
        out_ref[...] = h

    in_specs = [
        pl.BlockSpec((NC, block_rows, d), lambda i: (0, i, 0)),
        pl.BlockSpec((d, d), lambda i: (0, 0)),
        pl.BlockSpec((1, d), lambda i: (0, 0)),
    ]
    args = [parts, w_mat, bias.reshape(1, d)]
    if has_skip:
        in_specs.append(pl.BlockSpec((block_rows, d), lambda i: (i, 0)))
        args.append(skip)

    return pl.pallas_call(
        body,
        grid=(grid,),
        in_specs=in_specs,
        out_specs=pl.BlockSpec((block_rows, d), lambda i: (i, 0)),
        out_shape=jax.ShapeDtypeStruct((n1, d), jnp.float32),
    )(*args)


def _layer_norm(h, gamma, beta, block_rows=1000):
    n, d = h.shape
    grid = n // block_rows

    def body(h_ref, g_ref, b_ref, out_ref):
        x = h_ref[...]
        mu = jnp.mean(x, axis=1, keepdims=True)
        var = jnp.mean((x - mu) ** 2, axis=1, keepdims=True)
        out_ref[...] = (x - mu) * lax.rsqrt(var + 1e-5) * g_ref[...] + b_ref[...]

    return pl.pallas_call(
        body,
        grid=(grid,),
        in_specs=[
            pl.BlockSpec((block_rows, d), lambda i: (i, 0)),
            pl.BlockSpec((1, d), lambda i: (0, 0)),
            pl.BlockSpec((1, d), lambda i: (0, 0)),
        ],
        out_specs=pl.BlockSpec((block_rows, d), lambda i: (i, 0)),
        out_shape=jax.ShapeDtypeStruct((n, d), jnp.float32),
    )(h, gamma.reshape(1, d), beta.reshape(1, d))


def kernel(feat, edge_index, W0, b0, Ws, bs, W1, b1, gamma, beta):
    n, d = feat.shape
    e = edge_index.shape[1]

    n1 = -(-(n + 1) // 1264) * 1264

    kt = -(-e // (NS * CH))
    k0 = int(kt * K0_FRAC + 0.5)
    k1 = kt - k0
    kmax = max(k0, k1)

    src = edge_index[0]
    dst = edge_index[1]
    pad = NS * kt * CH - e
    src_p = jnp.concatenate([src, jnp.zeros((pad,), jnp.int32)])
    dst_p = jnp.concatenate([dst, jnp.full((pad,), n, jnp.int32)])

    def split(flat, fill):
        per_tile = flat.reshape(NS, kt, CH)
        c0 = per_tile[:, :k0]
        c1 = per_tile[:, k0:]
        if k0 < kmax:
            c0 = jnp.concatenate(
                [c0, jnp.full((NS, kmax - k0, CH), fill, jnp.int32)], axis=1)
        if k1 < kmax:
            c1 = jnp.concatenate(
                [c1, jnp.full((NS, kmax - k1, CH), fill, jnp.int32)], axis=1)
        return jnp.stack([c0, c1], axis=1).reshape(NW, kmax, CH)

    src_t = split(src_p, 0)
    dst_t = split(dst_p, n)
    zeros_slab = jnp.zeros((n1 // NS, d), jnp.float32)

    parts = _spmm_sc(feat, src_t, dst_t, zeros_slab, n1, k0, k1)
    h = _dense_layer(parts, W0, b0, None)
    for _ in range(3):
        parts = _spmm_sc(h, src_t, dst_t, zeros_slab, n1, k0, k1)
        h = _dense_layer(parts, Ws, bs, h)
    parts = _spmm_sc(h, src_t, dst_t, zeros_slab, n1, k0, k1)
    h = _dense_layer(parts, W1, b1, None)
    return _layer_norm(h[:n], gamma, beta)


# speedup vs baseline: 1.3175x; 1.0403x over previous
import functools

import jax
import jax.numpy as jnp
from jax import lax
from jax.experimental import pallas as pl
from jax.experimental.pallas import tpu as pltpu
from jax.experimental.pallas import tpu_sc as plsc

NC = 2
NS = 16
NW = NC * NS
CH = 128
K0_FRAC = 0.57


def _spmm_sc(z, src_t, dst_t, zeros_slab, n1, k0, k1):
    d = z.shape[1]
    rt = n1 // NS
    kmax = src_t.shape[1]

    mesh = plsc.VectorSubcoreMesh(core_axis_name="c", subcore_axis_name="s")

    @functools.partial(
        pl.kernel,
        out_type=jax.ShapeDtypeStruct((NC, n1, d), jnp.float32),
        mesh=mesh,
        scratch_types=[
            pltpu.VMEM((kmax, CH), jnp.int32),
            pltpu.VMEM((kmax, CH), jnp.int32),
            pltpu.VMEM((CH, d), jnp.float32),
            pltpu.VMEM_SHARED((n1, d), jnp.float32),
        ],
    )
    def spmm(z_hbm, src_hbm, dst_hbm, zeros_hbm, out_hbm,
             idx_s, idx_d, buf, acc):
        c = lax.axis_index("c")
        s = lax.axis_index("s")
        w = s * NC + c
        my_k = jnp.where(c == 0, k0, k1)

        pltpu.sync_copy(zeros_hbm, acc.at[pl.ds(s * rt, rt)])
        pltpu.sync_copy(src_hbm.at[w], idx_s)
        pltpu.sync_copy(dst_hbm.at[w], idx_d)
        plsc.subcore_barrier()

        def body(j, carry):
            pltpu.sync_copy(z_hbm.at[idx_s.at[j]], buf)
            pltpu.sync_copy(buf, acc.at[idx_d.at[j]], add=True)
            return carry

        lax.fori_loop(0, my_k, body, 0)
        plsc.subcore_barrier()
        pltpu.sync_copy(acc.at[pl.ds(s * rt, rt)],
                        out_hbm.at[c, pl.ds(s * rt, rt)])

    return spmm(z, src_t, dst_t, zeros_slab)


def _dense_layer(parts, w_mat, bias, skip, block_rows=1264):
    n1, d = parts.shape[1], parts.shape[2]
    grid = n1 // block_rows
    has_skip = skip is not None

    def body(p_ref, w_ref, b_ref, *rest):
        if has_skip:
            skip_ref, out_ref = rest
        else:
            (out_ref,) = rest
        agg = p_ref[0] + p_ref[1]
        h = jnp.maximum(
            jnp.dot(agg, w_ref[...], preferred_element_type=jnp.float32)
            + b_ref[...], 0.0)
        if has_skip:
            h = h + skip_ref[...]
        out_ref[...] = h

    in_specs = [
        pl.BlockSpec((NC, block_rows, d), lambda i: (0, i, 0)),
        pl.BlockSpec((d, d), lambda i: (0, 0)),
        pl.BlockSpec((1, d), lambda i: (0, 0)),
    ]
    args = [parts, w_mat, bias.reshape(1, d)]
    if has_skip:
        in_specs.append(pl.BlockSpec((block_rows, d), lambda i: (i, 0)))
        args.append(skip)

    return pl.pallas_call(
        body,
        grid=(grid,),
        in_specs=in_specs,
        out_specs=pl.BlockSpec((block_rows, d), lambda i: (i, 0)),
        out_shape=jax.ShapeDtypeStruct((n1, d), jnp.float32),
    )(*args)


def _layer_norm(h, gamma, beta, block_rows=1000):
    n, d = h.shape
    grid = n // block_rows

    def body(h_ref, g_ref, b_ref, out_ref):
        x = h_ref[...]
        mu = jnp.mean(x, axis=1, keepdims=True)
        var = jnp.mean((x - mu) ** 2, axis=1, keepdims=True)
        out_ref[...] = (x - mu) * lax.rsqrt(var + 1e-5) * g_ref[...] + b_ref[...]

    return pl.pallas_call(
        body,
        grid=(grid,),
        in_specs=[
            pl.BlockSpec((block_rows, d), lambda i: (i, 0)),
            pl.BlockSpec((1, d), lambda i: (0, 0)),
            pl.BlockSpec((1, d), lambda i: (0, 0)),
        ],
        out_specs=pl.BlockSpec((block_rows, d), lambda i: (i, 0)),
        out_shape=jax.ShapeDtypeStruct((n, d), jnp.float32),
    )(h, gamma.reshape(1, d), beta.reshape(1, d))


def kernel(feat, edge_index, W0, b0, Ws, bs, W1, b1, gamma, beta):
    n, d = feat.shape
    e = edge_index.shape[1]

    n1 = -(-(n + 1) // 1264) * 1264

    kt = -(-e // (NS * CH))
    k0 = int(kt * K0_FRAC + 0.5)
    k1 = kt - k0
    kmax = max(k0, k1)

    src = edge_index[0]
    dst = edge_index[1]
    pad = NS * kt * CH - e
    src_p = jnp.concatenate([src, jnp.zeros((pad,), jnp.int32)])
    dst_p = jnp.concatenate([dst, jnp.full((pad,), n, jnp.int32)])

    def split(flat, fill):
        per_tile = flat.reshape(NS, kt, CH)
        c0 = per_tile[:, :k0]
        c1 = per_tile[:, k0:]
        if k0 < kmax:
            c0 = jnp.concatenate(
                [c0, jnp.full((NS, kmax - k0, CH), fill, jnp.int32)], axis=1)
        if k1 < kmax:
            c1 = jnp.concatenate(
                [c1, jnp.full((NS, kmax - k1, CH), fill, jnp.int32)], axis=1)
        return jnp.stack([c0, c1], axis=1).reshape(NW, kmax, CH)

    src_t = split(src_p, 0)
    dst_t = split(dst_p, n)
    zeros_slab = jnp.zeros((n1 // NS, d), jnp.float32)

    parts = _spmm_sc(feat, src_t, dst_t, zeros_slab, n1, k0, k1)
    h = _dense_layer(parts, W0, b0, None)
    for _ in range(3):
        parts = _spmm_sc(h, src_t, dst_t, zeros_slab, n1, k0, k1)
        h = _dense_layer(parts, Ws, bs, h)
    parts = _spmm_sc(h, src_t, dst_t, zeros_slab, n1, k0, k1)
    h = _dense_layer(parts, W1, b1, None)
    return _layer_norm(h[:n], gamma, beta)


# speedup vs baseline: 1.3460x; 1.0216x over previous
import functools

import jax
import jax.numpy as jnp
from jax import lax
from jax.experimental import pallas as pl
from jax.experimental.pallas import tpu as pltpu
from jax.experimental.pallas import tpu_sc as plsc

NC = 2
NS = 16
NW = NC * NS
CH = 128
K0_FRAC = 0.60


def _spmm_sc(z, src_t, dst_t, zeros_slab, n1, k0, k1):
    d = z.shape[1]
    rt = n1 // NS
    kmax = src_t.shape[1]

    mesh = plsc.VectorSubcoreMesh(core_axis_name="c", subcore_axis_name="s")

    @functools.partial(
        pl.kernel,
        out_type=jax.ShapeDtypeStruct((NC, n1, d), jnp.float32),
        mesh=mesh,
        scratch_types=[
            pltpu.VMEM((kmax, CH), jnp.int32),
            pltpu.VMEM((kmax, CH), jnp.int32),
            pltpu.VMEM((CH, d), jnp.float32),
            pltpu.VMEM_SHARED((n1, d), jnp.float32),
        ],
    )
    def spmm(z_hbm, src_hbm, dst_hbm, zeros_hbm, out_hbm,
             idx_s, idx_d, buf, acc):
        c = lax.axis_index("c")
        s = lax.axis_index("s")
        w = s * NC + c
        my_k = jnp.where(c == 0, k0, k1)

        pltpu.sync_copy(zeros_hbm, acc.at[pl.ds(s * rt, rt)])
        pltpu.sync_copy(src_hbm.at[w], idx_s)
        pltpu.sync_copy(dst_hbm.at[w], idx_d)
        plsc.subcore_barrier()

        def body(j, carry):
            pltpu.sync_copy(z_hbm.at[idx_s.at[j]], buf)
            pltpu.sync_copy(buf, acc.at[idx_d.at[j]], add=True)
            return carry

        lax.fori_loop(0, my_k, body, 0)
        plsc.subcore_barrier()
        pltpu.sync_copy(acc.at[pl.ds(s * rt, rt)],
                        out_hbm.at[c, pl.ds(s * rt, rt)])

    return spmm(z, src_t, dst_t, zeros_slab)


def _dense_layer(parts, w_mat, bias, skip, block_rows=1264):
    n1, d = parts.shape[1], parts.shape[2]
    grid = n1 // block_rows
    has_skip = skip is not None

    def body(p_ref, w_ref, b_ref, *rest):
        if has_skip:
            skip_ref, out_ref = rest
        else:
            (out_ref,) = rest
        agg = p_ref[0] + p_ref[1]
        h = jnp.maximum(
            jnp.dot(agg, w_ref[...], preferred_element_type=jnp.float32)
            + b_ref[...], 0.0)
        if has_skip:
            h = h + skip_ref[...]
        out_ref[...] = h

    in_specs = [
        pl.BlockSpec((NC, block_rows, d), lambda i: (0, i, 0)),
        pl.BlockSpec((d, d), lambda i: (0, 0)),
        pl.BlockSpec((1, d), lambda i: (0, 0)),
    ]
    args = [parts, w_mat, bias.reshape(1, d)]
    if has_skip:
        in_specs.append(pl.BlockSpec((block_rows, d), lambda i: (i, 0)))
        args.append(skip)

    return pl.pallas_call(
        body,
        grid=(grid,),
        in_specs=in_specs,
        out_specs=pl.BlockSpec((block_rows, d), lambda i: (i, 0)),
        out_shape=jax.ShapeDtypeStruct((n1, d), jnp.float32),
    )(*args)


def _layer_norm(h, gamma, beta, block_rows=1000):
    n, d = h.shape
    grid = n // block_rows

    def body(h_ref, g_ref, b_ref, out_ref):
        x = h_ref[...]
        mu = jnp.mean(x, axis=1, keepdims=True)
        var = jnp.mean((x - mu) ** 2, axis=1, keepdims=True)
        out_ref[...] = (x - mu) * lax.rsqrt(var + 1e-5) * g_ref[...] + b_ref[...]

    return pl.pallas_call(
        body,
        grid=(grid,),
        in_specs=[
            pl.BlockSpec((block_rows, d), lambda i: (i, 0)),
            pl.BlockSpec((1, d), lambda i: (0, 0)),
            pl.BlockSpec((1, d), lambda i: (0, 0)),
        ],
        out_specs=pl.BlockSpec((block_rows, d), lambda i: (i, 0)),
        out_shape=jax.ShapeDtypeStruct((n, d), jnp.float32),
    )(h, gamma.reshape(1, d), beta.reshape(1, d))


def kernel(feat, edge_index, W0, b0, Ws, bs, W1, b1, gamma, beta):
    n, d = feat.shape
    e = edge_index.shape[1]

    n1 = -(-(n + 1) // 1264) * 1264

    kt = -(-e // (NS * CH))
    k0 = int(kt * K0_FRAC + 0.5)
    k1 = kt - k0
    kmax = max(k0, k1)

    src = edge_index[0]
    dst = edge_index[1]
    pad = NS * kt * CH - e
    src_p = jnp.concatenate([src, jnp.zeros((pad,), jnp.int32)])
    dst_p = jnp.concatenate([dst, jnp.full((pad,), n, jnp.int32)])

    def split(flat, fill):
        per_tile = flat.reshape(NS, kt, CH)
        c0 = per_tile[:, :k0]
        c1 = per_tile[:, k0:]
        if k0 < kmax:
            c0 = jnp.concatenate(
                [c0, jnp.full((NS, kmax - k0, CH), fill, jnp.int32)], axis=1)
        if k1 < kmax:
            c1 = jnp.concatenate(
                [c1, jnp.full((NS, kmax - k1, CH), fill, jnp.int32)], axis=1)
        return jnp.stack([c0, c1], axis=1).reshape(NW, kmax, CH)

    src_t = split(src_p, 0)
    dst_t = split(dst_p, n)
    zeros_slab = jnp.zeros((n1 // NS, d), jnp.float32)

    parts = _spmm_sc(feat, src_t, dst_t, zeros_slab, n1, k0, k1)
    h = _dense_layer(parts, W0, b0, None)
    for _ in range(3):
        parts = _spmm_sc(h, src_t, dst_t, zeros_slab, n1, k0, k1)
        h = _dense_layer(parts, Ws, bs, h)
    parts = _spmm_sc(h, src_t, dst_t, zeros_slab, n1, k0, k1)
    h = _dense_layer(parts, W1, b1, None)
    return _layer_norm(h[:n], gamma, beta)
